# baseline (device time: 48788 ns/iter reference)
import jax
import jax.numpy as jnp
from jax import lax
from jax.experimental import pallas as pl
from jax.experimental.pallas import tpu as pltpu

N_DEV = 4


def kernel(x, k, Wp):
    B, S, C = x.shape
    K = k.shape[0]
    N = Wp.shape[1]
    M = B * S
    H = M // 2
    Q = H // 2
    E = Q // 2
    F = E // 2

    def body(x_hbm, k_hbm, w_hbm, out_hbm, acc_ref, sbuf_ref, rbuf_ref,
             out_ref, xb0_ref, xb1_ref, kw_ref, ww_ref,
             send_sems, recv_sems, dma_sems):
        my = lax.axis_index("i")
        x_b = my // 2
        y_b = (my // 2) ^ (my % 2)
        py = my ^ 1
        px = (N_DEV - 1) - my

        b_send0, b_send1 = 1 - y_b, 2 + (1 - x_b)
        b_kept0, b_kept1 = y_b, 2 + x_b
        cp_k = pltpu.make_async_copy(k_hbm, kw_ref, dma_sems.at[0])
        cp_k.start()
        cp_w = pltpu.make_async_copy(w_hbm, ww_ref, dma_sems.at[1])
        cp_w.start()
        cp_x0 = pltpu.make_async_copy(
            x_hbm.at[pl.ds(b_send0, 1)], xb0_ref, dma_sems.at[2])
        cp_x0.start()
        cp_x1 = pltpu.make_async_copy(
            x_hbm.at[pl.ds(b_send1, 1)], xb1_ref, dma_sems.at[3])
        cp_x1.start()

        barrier_sem = pltpu.get_barrier_semaphore()
        for nbr in (py, px):
            pl.semaphore_signal(
                barrier_sem, inc=1,
                device_id=(nbr,), device_id_type=pl.DeviceIdType.MESH,
            )

        cp_k.wait()
        cp_w.wait()
        kb = kw_ref[:, :].astype(jnp.bfloat16)
        w = ww_ref[:, :].astype(jnp.bfloat16)

        def partial_batch(buf_ref):
            xb = buf_ref[:, :, :].astype(jnp.bfloat16)
            conv = xb * kb[K - 1, :]
            for t in range(K - 1):
                shift = K - 1 - t
                shifted = jnp.concatenate(
                    [jnp.zeros((1, shift, C), dtype=xb.dtype),
                     xb[:, : S - shift, :]],
                    axis=1,
                )
                conv = conv + shifted * kb[t, :]
            cf = conv.astype(jnp.float32)
            a = cf * (1.0 / (1.0 + jnp.exp(-cf)))
            a2 = a.reshape(S, C).astype(jnp.bfloat16)
            return jnp.dot(a2, w, preferred_element_type=jnp.float32)

        rdmas = []

        def exchange(src_ref, dst_ref, sem_idx, partner):
            rdma = pltpu.make_async_remote_copy(
                src_ref=src_ref,
                dst_ref=dst_ref,
                send_sem=send_sems.at[sem_idx],
                recv_sem=recv_sems.at[sem_idx],
                device_id=(partner,),
                device_id_type=pl.DeviceIdType.MESH,
            )
            rdma.start()
            rdmas.append(rdma)
            return rdma

        k0 = y_b * Q
        k1 = H + x_b * Q
        s1_0 = (1 - y_b) * Q
        s1_1 = H + (1 - x_b) * Q
        s2_0 = k0 + (1 - x_b) * E
        s2_1 = k1 + (1 - y_b) * E
        k2_0 = k0 + x_b * E
        k2_1 = k1 + y_b * E

        c1a = (1 - x_b) * E
        c2a = x_b * E
        c1b = (1 - y_b) * E
        c2b = y_b * E

        cp_x0.wait()
        sbuf_ref[pl.ds(s1_0, Q), :] = partial_batch(xb0_ref).astype(
            jnp.bfloat16)
        pl.semaphore_wait(barrier_sem, 2)
        p1a_1 = exchange(sbuf_ref.at[pl.ds(s1_0 + c1a, F)],
                         rbuf_ref.at[pl.ds(c1a, F)], 0, py)
        p1a_2 = exchange(sbuf_ref.at[pl.ds(s1_0 + c1a + F, F)],
                         rbuf_ref.at[pl.ds(c1a + F, F)], 2, py)
        p1a_3 = exchange(sbuf_ref.at[pl.ds(s1_0 + c2a, E)],
                         rbuf_ref.at[pl.ds(c2a, E)], 4, py)
        cp_x1.wait()
        cp_x2 = pltpu.make_async_copy(
            x_hbm.at[pl.ds(b_kept0, 1)], xb0_ref, dma_sems.at[4])
        cp_x2.start()
        sbuf_ref[pl.ds(s1_1, Q), :] = partial_batch(xb1_ref).astype(
            jnp.bfloat16)
        p1b_1 = exchange(sbuf_ref.at[pl.ds(s1_1 + c1b, F)],
                         rbuf_ref.at[pl.ds(Q + c1b, F)], 1, px)
        p1b_2 = exchange(sbuf_ref.at[pl.ds(s1_1 + c1b + F, F)],
                         rbuf_ref.at[pl.ds(Q + c1b + F, F)], 3, px)
        p1b_3 = exchange(sbuf_ref.at[pl.ds(s1_1 + c2b, E)],
                         rbuf_ref.at[pl.ds(Q + c2b, E)], 5, px)
        cp_x3 = pltpu.make_async_copy(
            x_hbm.at[pl.ds(b_kept1, 1)], xb1_ref, dma_sems.at[5])

        cp_x2.wait()
        cp_x3.start()
        acc_ref[pl.ds(k0, Q), :] = partial_batch(xb0_ref)
        cp_x3.wait()
        acc_ref[pl.ds(k1, Q), :] = partial_batch(xb1_ref)

        def fuse2(dst, a_off, r_off, n):
            sbuf_ref[pl.ds(dst, n), :] = (
                acc_ref[pl.ds(a_off, n), :]
                + rbuf_ref[pl.ds(r_off, n), :].astype(jnp.float32)
            ).astype(jnp.bfloat16)

        p1a_1.wait_recv()
        fuse2(s2_0, s2_0, c1a, F)
        p2a_1 = exchange(sbuf_ref.at[pl.ds(s2_0, F)],
                         rbuf_ref.at[pl.ds(2 * Q, F)], 6, px)
        p1b_1.wait_recv()
        fuse2(s2_1, s2_1, Q + c1b, F)
        p2b_1 = exchange(sbuf_ref.at[pl.ds(s2_1, F)],
                         rbuf_ref.at[pl.ds(2 * Q + E, F)], 7, py)
        p1a_2.wait_recv()
        fuse2(s2_0 + F, s2_0 + F, c1a + F, F)
        p2a_2 = exchange(sbuf_ref.at[pl.ds(s2_0 + F, F)],
                         rbuf_ref.at[pl.ds(2 * Q + F, F)], 8, px)
        p1b_2.wait_recv()
        fuse2(s2_1 + F, s2_1 + F, Q + c1b + F, F)
        p2b_2 = exchange(sbuf_ref.at[pl.ds(s2_1 + F, F)],
                         rbuf_ref.at[pl.ds(2 * Q + E + F, F)], 9, py)

        p1a_3.wait_recv()
        p1b_3.wait_recv()

        def outwrite(bat, seq, a_off, r1_off, r2_off):
            out_ref[pl.ds(bat, 1), pl.ds(seq, F), :] = (
                acc_ref[pl.ds(a_off, F), :]
                + rbuf_ref[pl.ds(r1_off, F), :].astype(jnp.float32)
                + rbuf_ref[pl.ds(r2_off, F), :].astype(jnp.float32)
            ).astype(jnp.bfloat16).reshape(1, F, N)

        p2b_1.wait_recv()
        outwrite(2 + x_b, y_b * E, k2_1, Q + c2b, 2 * Q + E)
        o3b1 = out_ref.at[pl.ds(2 + x_b, 1), pl.ds(y_b * E, F)]
        p3b_1 = exchange(o3b1, o3b1, 11, py)
        p2a_1.wait_recv()
        outwrite(y_b, x_b * E, k2_0, c2a, 2 * Q)
        o3a1 = out_ref.at[pl.ds(y_b, 1), pl.ds(x_b * E, F)]
        p3a_1 = exchange(o3a1, o3a1, 10, px)
        p2b_2.wait_recv()
        outwrite(2 + x_b, y_b * E + F, k2_1 + F, Q + c2b + F, 2 * Q + E + F)
        o3b2 = out_ref.at[pl.ds(2 + x_b, 1), pl.ds(y_b * E + F, F)]
        p3b_2 = exchange(o3b2, o3b2, 13, py)
        p2a_2.wait_recv()
        outwrite(y_b, x_b * E + F, k2_0 + F, c2a + F, 2 * Q + F)
        o3a2 = out_ref.at[pl.ds(y_b, 1), pl.ds(x_b * E + F, F)]
        p3a_2 = exchange(o3a2, o3a2, 12, px)

        o4a = out_ref.at[pl.ds(y_b, 1), pl.ds(x_b * E, E)]
        p4a_own = exchange(o4a, o4a, 14, py)
        o4b = out_ref.at[pl.ds(2 + x_b, 1), pl.ds(y_b * E, E)]
        p4b_own = exchange(o4b, o4b, 15, px)

        p3b_1.wait_recv()
        o4bg1 = out_ref.at[pl.ds(2 + x_b, 1), pl.ds((1 - y_b) * E, F)]
        p4b_g1 = exchange(o4bg1, o4bg1, 17, px)
        p3a_1.wait_recv()
        o4ag1 = out_ref.at[pl.ds(y_b, 1), pl.ds((1 - x_b) * E, F)]
        p4a_g1 = exchange(o4ag1, o4ag1, 16, py)
        p3b_2.wait_recv()
        o4bg2 = out_ref.at[pl.ds(2 + x_b, 1), pl.ds((1 - y_b) * E + F, F)]
        p4b_g2 = exchange(o4bg2, o4bg2, 19, px)
        cp_o1 = pltpu.make_async_copy(
            out_ref.at[pl.ds(2 + x_b, 1)], out_hbm.at[pl.ds(2 + x_b, 1)],
            dma_sems.at[6])
        cp_o1.start()
        p3a_2.wait_recv()
        o4ag2 = out_ref.at[pl.ds(y_b, 1), pl.ds((1 - x_b) * E + F, F)]
        p4a_g2 = exchange(o4ag2, o4ag2, 18, py)
        cp_o0 = pltpu.make_async_copy(
            out_ref.at[pl.ds(y_b, 1)], out_hbm.at[pl.ds(y_b, 1)],
            dma_sems.at[7])
        cp_o0.start()

        for r in (p4a_own, p4a_g1, p4a_g2):
            r.wait_recv()
        cp_o2 = pltpu.make_async_copy(
            out_ref.at[pl.ds(1 - y_b, 1)], out_hbm.at[pl.ds(1 - y_b, 1)],
            dma_sems.at[8])
        cp_o2.start()
        for r in (p4b_own, p4b_g1, p4b_g2):
            r.wait_recv()
        cp_o3 = pltpu.make_async_copy(
            out_ref.at[pl.ds(2 + (1 - x_b), 1)],
            out_hbm.at[pl.ds(2 + (1 - x_b), 1)],
            dma_sems.at[9])
        cp_o3.start()

        for cp in (cp_o1, cp_o0, cp_o2, cp_o3):
            cp.wait()
        for r in rdmas:
            r.wait_send()

    return pl.pallas_call(
        body,
        out_shape=jax.ShapeDtypeStruct((B, S, N), jnp.bfloat16),
        in_specs=[
            pl.BlockSpec(memory_space=pltpu.MemorySpace.HBM),
            pl.BlockSpec(memory_space=pltpu.MemorySpace.HBM),
            pl.BlockSpec(memory_space=pltpu.MemorySpace.HBM),
        ],
        out_specs=pl.BlockSpec(memory_space=pltpu.MemorySpace.HBM),
        scratch_shapes=[
            pltpu.VMEM((M, N), jnp.float32),
            pltpu.VMEM((M, N), jnp.bfloat16),
            pltpu.VMEM((3 * M // 4, N), jnp.bfloat16),
            pltpu.VMEM((B, S, N), jnp.bfloat16),
            pltpu.VMEM((1, S, C), jnp.float32),
            pltpu.VMEM((1, S, C), jnp.float32),
            pltpu.VMEM((K, C), jnp.float32),
            pltpu.VMEM((C, N), jnp.float32),
            pltpu.SemaphoreType.DMA((20,)),
            pltpu.SemaphoreType.DMA((20,)),
            pltpu.SemaphoreType.DMA((10,)),
        ],
        compiler_params=pltpu.CompilerParams(collective_id=0),
    )(x, k, Wp)
